# baseline (speedup 1.0000x reference)
import functools

import jax
import jax.numpy as jnp
from jax import lax
from jax.experimental import pallas as pl
from jax.experimental.pallas import tpu as pltpu
from jax.experimental.pallas import tpu_sc as plsc

NC = 2
NS = 16
NW = NC * NS
LANES = 16
CH = 128


def _prep_body(pos_ref, seg_tab_ref, pose_ref, d_ref):
    pos = pos_ref[...]
    seg = seg_tab_ref[...]
    seg0 = seg[0:1, :]
    ext = jnp.concatenate([pos, pos[:CH, :]], axis=0)
    pose_ref[...] = ext + seg0
    d1 = seg[1:2, :] - seg0
    d2 = seg[2:3, :] - seg0
    a_row = 2.0 * d1 - 0.5 * d2
    b_row = 0.5 * d2 - d1
    d_ref[...] = jnp.concatenate([a_row, b_row], axis=0)


def _sc_body(n_chunks, seq_len, tok_hbm, ids_hbm, seg_hbm, pose_hbm, d_hbm,
             out_hbm, idx_t, seg_v, pose_v, d_v, rt0, rt1, gs0, gs1, ss0, ss1):
    wid = lax.axis_index("s") * NC + lax.axis_index("c")
    pw = n_chunks * CH
    base0 = wid * pw

    pltpu.sync_copy(pose_hbm, pose_v)
    pltpu.sync_copy(d_hbm, d_v)
    pltpu.sync_copy(ids_hbm.at[pl.ds(base0, pw)], idx_t)
    pltpu.sync_copy(seg_hbm.at[pl.ds(base0, pw)], seg_v)

    d1c = [d_v[0, pl.ds(c * LANES, LANES)] for c in range(8)]
    d2c = [d_v[1, pl.ds(c * LANES, LANES)] for c in range(8)]

    def fire(g, rt, gsem):
        it = idx_t.at[pl.ds(g * CH, CH)]
        pltpu.async_copy(tok_hbm.at[it], rt, gsem)

    def wait_scatter(rt, ssem):
        pltpu.make_async_copy(rt, out_hbm.at[pl.ds(base0, CH)], ssem).wait()

    def proc(g, rt, gsem, ssem):
        pltpu.make_async_copy(tok_hbm.at[pl.ds(0, CH)], rt, gsem).wait()
        p0 = (base0 + g * CH) % seq_len

        def block_body(rb, rcarry):
            r0 = rb * LANES
            cvec = seg_v[pl.ds(g * CH + r0, LANES)].astype(jnp.float32)
            for j in range(LANES):
                r = r0 + j
                pr = p0 + r
                sv = jnp.broadcast_to(cvec[j], (LANES,))
                sv2 = sv * sv
                for c in range(8):
                    sl = pl.ds(c * LANES, LANES)
                    v = rt[r, sl] + pose_v[pr, sl]
                    v = v + sv * d1c[c] + sv2 * d2c[c]
                    rt[r, sl] = v
            return rcarry

        lax.fori_loop(0, CH // LANES, block_body, 0)
        pltpu.async_copy(rt, out_hbm.at[pl.ds(base0 + g * CH, CH)], ssem)

    n_pairs = n_chunks // 2
    fire(0, rt0, gs0)
    fire(1, rt1, gs1)

    def pair_body(k, carry):
        g0 = 2 * k
        proc(g0, rt0, gs0, ss0)

        @pl.when(k < n_pairs - 1)
        def _():
            wait_scatter(rt0, ss0)
            fire(g0 + 2, rt0, gs0)

        proc(g0 + 1, rt1, gs1, ss1)

        @pl.when(k < n_pairs - 1)
        def _():
            wait_scatter(rt1, ss1)
            fire(g0 + 3, rt1, gs1)

        return carry

    lax.fori_loop(0, n_pairs, pair_body, 0)
    wait_scatter(rt0, ss0)
    wait_scatter(rt1, ss1)


def kernel(input_ids, segment_ids, token_table, segment_table, pos_table):
    B, S = input_ids.shape
    H = token_table.shape[1]
    R = B * S
    assert R % (NW * CH * 2) == 0
    n_chunks = R // (NW * CH)

    pose, d = pl.pallas_call(
        _prep_body,
        out_shape=(
            jax.ShapeDtypeStruct((S + CH, H), jnp.float32),
            jax.ShapeDtypeStruct((2, H), jnp.float32),
        ),
    )(pos_table[:S], segment_table)

    ids_flat = input_ids.astype(jnp.int32).reshape(R)
    seg_flat = segment_ids.astype(jnp.int32).reshape(R)

    sc_fn = functools.partial(
        pl.kernel,
        out_type=jax.ShapeDtypeStruct((R, H), jnp.float32),
        mesh=plsc.VectorSubcoreMesh(core_axis_name="c", subcore_axis_name="s"),
        scratch_types=[
            pltpu.VMEM((R // NW,), jnp.int32),
            pltpu.VMEM((R // NW,), jnp.int32),
            pltpu.VMEM((S + CH, H), jnp.float32),
            pltpu.VMEM((2, H), jnp.float32),
            pltpu.VMEM((CH, H), jnp.float32),
            pltpu.VMEM((CH, H), jnp.float32),
            pltpu.SemaphoreType.DMA,
            pltpu.SemaphoreType.DMA,
            pltpu.SemaphoreType.DMA,
            pltpu.SemaphoreType.DMA,
        ],
    )(functools.partial(_sc_body, n_chunks, S))

    out2d = sc_fn(token_table, ids_flat, seg_flat, pose, d)
    return out2d.reshape(B, S, H)


# speedup vs baseline: 2.2364x; 2.2364x over previous
import functools

import jax
import jax.numpy as jnp
from jax import lax
from jax.experimental import pallas as pl
from jax.experimental.pallas import tpu as pltpu
from jax.experimental.pallas import tpu_sc as plsc

NC = 2
NS = 16
NW = NC * NS
LANES = 16
CH = 128


def _prep_body(seg_ids_ref, pos_ref, seg_tab_ref, comb_ref, cidx_ref):
    comb_ref[...] = pos_ref[...][:, None, :] + seg_tab_ref[...][None, :, :]
    s_iota = lax.broadcasted_iota(jnp.int32, seg_ids_ref.shape, 1)
    cidx_ref[...] = seg_ids_ref[...] + 3 * s_iota


def _sc_body(n_chunks, tok_hbm, ids_hbm, cidx_hbm, comb_hbm, out_hbm,
             idx_t, idx_c, rt0, rt1, rt2, ro0, ro1, ro2,
             gs0, gs1, gs2, ss0, ss1, ss2):
    wid = lax.axis_index("s") * NC + lax.axis_index("c")
    pw = n_chunks * CH
    base0 = wid * pw

    pltpu.sync_copy(ids_hbm.at[pl.ds(base0, pw)], idx_t)
    pltpu.sync_copy(cidx_hbm.at[pl.ds(base0, pw)], idx_c)

    def fire(g, rt, ro, gsem):
        it = idx_t.at[pl.ds(g * CH, CH)]
        ic = idx_c.at[pl.ds(g * CH, CH)]
        pltpu.async_copy(tok_hbm.at[it], rt, gsem)
        pltpu.async_copy(comb_hbm.at[ic], ro, gsem)

    def wait_scatter(ro, ssem):
        pltpu.make_async_copy(ro, out_hbm.at[pl.ds(base0, CH)], ssem).wait()

    def proc(g, rt, ro, gsem, ssem):
        dummy = tok_hbm.at[pl.ds(0, CH)]
        pltpu.make_async_copy(dummy, rt, gsem).wait()
        pltpu.make_async_copy(dummy, ro, gsem).wait()

        def row_body(r, rcarry):
            for c in range(8):
                sl = pl.ds(c * LANES, LANES)
                plsc.addupdate(ro.at[r, sl], rt[r, sl])
            return rcarry

        lax.fori_loop(0, CH, row_body, 0)
        pltpu.async_copy(ro, out_hbm.at[pl.ds(base0 + g * CH, CH)], ssem)

    slots = ((rt0, ro0, gs0, ss0), (rt1, ro1, gs1, ss1), (rt2, ro2, gs2, ss2))
    nslots = len(slots)
    n_main = n_chunks // nslots
    n_tail = n_chunks - n_main * nslots

    for i, (rt, ro, gsem, _) in enumerate(slots):
        fire(i, rt, ro, gsem)

    def round_body(k, carry):
        g0 = nslots * k
        for i, (rt, ro, gsem, ssem) in enumerate(slots):
            proc(g0 + i, rt, ro, gsem, ssem)
            gn = g0 + i + nslots

            @pl.when(gn < n_chunks)
            def _(rt=rt, ro=ro, gsem=gsem, ssem=ssem, gn=gn):
                wait_scatter(ro, ssem)
                fire(gn, rt, ro, gsem)

        return carry

    lax.fori_loop(0, n_main, round_body, 0)
    for i in range(n_tail):
        rt, ro, gsem, ssem = slots[i]
        proc(n_main * nslots + i, rt, ro, gsem, ssem)
    for _, ro, _, ssem in slots:
        wait_scatter(ro, ssem)


def kernel(input_ids, segment_ids, token_table, segment_table, pos_table):
    B, S = input_ids.shape
    H = token_table.shape[1]
    R = B * S
    assert R % (NW * CH) == 0
    n_chunks = R // (NW * CH)

    comb3, cidx = pl.pallas_call(
        _prep_body,
        out_shape=(
            jax.ShapeDtypeStruct((S, 3, H), jnp.float32),
            jax.ShapeDtypeStruct((B, S), jnp.int32),
        ),
    )(segment_ids.astype(jnp.int32), pos_table[:S], segment_table)

    comb = comb3.reshape(S * 3, H)
    ids_flat = input_ids.astype(jnp.int32).reshape(R)
    cidx_flat = cidx.reshape(R)

    sc_fn = functools.partial(
        pl.kernel,
        out_type=jax.ShapeDtypeStruct((R, H), jnp.float32),
        mesh=plsc.VectorSubcoreMesh(core_axis_name="c", subcore_axis_name="s"),
        scratch_types=[
            pltpu.VMEM((R // NW,), jnp.int32),
            pltpu.VMEM((R // NW,), jnp.int32),
            pltpu.VMEM((CH, H), jnp.float32),
            pltpu.VMEM((CH, H), jnp.float32),
            pltpu.VMEM((CH, H), jnp.float32),
            pltpu.VMEM((CH, H), jnp.float32),
            pltpu.VMEM((CH, H), jnp.float32),
            pltpu.VMEM((CH, H), jnp.float32),
            pltpu.SemaphoreType.DMA,
            pltpu.SemaphoreType.DMA,
            pltpu.SemaphoreType.DMA,
            pltpu.SemaphoreType.DMA,
            pltpu.SemaphoreType.DMA,
            pltpu.SemaphoreType.DMA,
        ],
    )(functools.partial(_sc_body, n_chunks))

    out2d = sc_fn(token_table, ids_flat, cidx_flat, comb)
    return out2d.reshape(B, S, H)
